# baseline (speedup 1.0000x reference)
import functools

import jax
import jax.numpy as jnp
from jax import lax
from jax.experimental import pallas as pl
from jax.experimental.pallas import tpu as pltpu
from jax.experimental.pallas import tpu_sc as plsc

E = 8
K = 2
D = 1024
F = 2048
T = 2048
C = 640
NROWS = (E + 1) * C
DUMP = E * C
LANES = 128
NW = 32
CHUNK = 64
DCH = 32


def _router_body(x_ref, wg_ref, d0_ref, d1_ref, wrow_ref,
                 counts_ref, aux_ref):
    x = x_ref[...]
    wg = wg_ref[...]
    logits = jnp.dot(x, wg, preferred_element_type=jnp.float32)
    lane = lax.broadcasted_iota(jnp.int32, (T, LANES), 1).astype(jnp.float32)
    valid = lane < float(E)
    m = jnp.max(jnp.where(valid, logits, -jnp.inf), axis=1, keepdims=True)
    ex = jnp.where(valid, jnp.exp(logits - m), 0.0)
    z = jnp.sum(ex, axis=1, keepdims=True)
    probs = ex / z

    m1 = jnp.max(probs, axis=1, keepdims=True)
    is1 = jnp.logical_and(probs == m1, valid)
    i1 = jnp.min(jnp.where(is1, lane, float(LANES)), axis=1, keepdims=True)
    mask0 = (lane == i1).astype(jnp.float32)
    p2 = jnp.where(mask0 > 0, -1.0, probs)
    m2 = jnp.max(p2, axis=1, keepdims=True)
    is2 = jnp.logical_and(p2 == m2, valid)
    i2 = jnp.min(jnp.where(is2, lane, float(LANES)), axis=1, keepdims=True)
    mask1 = (lane == i2).astype(jnp.float32)

    denom = m1 + m2 + 1e-9
    g0 = m1 / denom
    g1 = m2 / denom

    B = 256
    r = lax.broadcasted_iota(jnp.int32, (B, B), 0)
    c = lax.broadcasted_iota(jnp.int32, (B, B), 1)
    ltri = (r > c).astype(jnp.float32)
    carry = jnp.zeros((1, LANES), dtype=jnp.float32)
    pos_parts = []
    for mask in (mask0, mask1):
        parts = []
        for b in range(T // B):
            mb = mask[b * B:(b + 1) * B, :]
            parts.append(jnp.dot(ltri, mb, preferred_element_type=jnp.float32)
                         + carry)
            carry = carry + jnp.sum(mb, axis=0, keepdims=True)
        pos_parts.append(jnp.concatenate(parts, axis=0))
    pos0, pos1 = pos_parts
    total = carry

    p0 = jnp.sum(pos0 * mask0, axis=1, keepdims=True)
    p1 = jnp.sum(pos1 * mask1, axis=1, keepdims=True)
    keep0 = p0 < float(C)
    keep1 = p1 < float(C)
    d0 = jnp.where(keep0, i1 * float(C) + p0, float(DUMP))
    d1 = jnp.where(keep1, i2 * float(C) + p1, float(DUMP))
    d0_ref[...] = d0.astype(jnp.int32)
    d1_ref[...] = d1.astype(jnp.int32)
    w0e = jnp.where(keep0, g0, 0.0)
    w1e = jnp.where(keep1, g1, 0.0)
    w_all = jnp.concatenate([w0e, w1e], axis=0)
    wrow_ref[...] = jnp.broadcast_to(w_all, (K * T, 128))
    counts_ref[...] = jnp.minimum(total, float(C))

    em = jnp.maximum(mask0, mask1)
    tpe = jnp.sum(em, axis=0, keepdims=True) * (1.0 / T)
    ppe = jnp.sum(probs, axis=0, keepdims=True) * (1.0 / T)
    aux_ref[...] = jnp.sum(tpe * ppe, axis=1, keepdims=True) * float(E)


def _run_router(x, wg_pad, interpret=False):
    out_shapes = (
        jax.ShapeDtypeStruct((T, 1), jnp.int32),
        jax.ShapeDtypeStruct((T, 1), jnp.int32),
        jax.ShapeDtypeStruct((K * T, 128), jnp.float32),
        jax.ShapeDtypeStruct((1, LANES), jnp.float32),
        jax.ShapeDtypeStruct((1, 1), jnp.float32),
    )
    return pl.pallas_call(
        _router_body,
        out_shape=out_shapes,
        interpret=interpret,
    )(x, wg_pad)


def _dispatch_body(x_hbm, d_hbm, wbb_hbm, ein_hbm, wslot_hbm,
                   idx_v, rows_v, wrows_v, sem, sem2):
    cid = lax.axis_index("c")
    sid = lax.axis_index("s")
    w = sid * 2 + cid
    t0 = (w % 16) * (T // 16)
    pltpu.sync_copy(d_hbm.at[w], idx_v)
    pltpu.sync_copy(wbb_hbm.at[w], wrows_v)
    for j in range(2):
        pltpu.sync_copy(x_hbm.at[pl.ds(t0 + j * CHUNK, CHUNK)], rows_v)
        cp1 = pltpu.async_copy(rows_v, ein_hbm.at[idx_v.at[j]], sem)
        cp2 = pltpu.async_copy(wrows_v.at[j], wslot_hbm.at[idx_v.at[j]], sem2)
        cp1.wait()
        cp2.wait()


def _run_dispatch(x, d_b, w_bb):
    mesh = plsc.VectorSubcoreMesh(core_axis_name="c", subcore_axis_name="s")
    kern = functools.partial(
        pl.kernel,
        out_type=(
            jax.ShapeDtypeStruct((NROWS, D), jnp.float32),
            jax.ShapeDtypeStruct((NROWS, 128), jnp.float32),
        ),
        mesh=mesh,
        scratch_types=[
            pltpu.VMEM((2, CHUNK), jnp.int32),
            pltpu.VMEM((CHUNK, D), jnp.float32),
            pltpu.VMEM((2, CHUNK, 128), jnp.float32),
            pltpu.SemaphoreType.DMA,
            pltpu.SemaphoreType.DMA,
        ],
    )
    return kern(_dispatch_body)(x, d_b, w_bb)


def _ffn_body(counts_ref, xin_ref, w1_ref, w2_ref, ws_ref, out_ref):
    e = pl.program_id(0)
    cnt = counts_ref[0, e]
    row = lax.broadcasted_iota(jnp.int32, (C, 1), 0).astype(jnp.float32)
    x = jnp.where(row < cnt, xin_ref[...], 0.0).astype(jnp.bfloat16)
    h = jnp.dot(x, w1_ref[0], preferred_element_type=jnp.float32)
    h = jax.nn.gelu(h, approximate=True).astype(jnp.bfloat16)
    out = jnp.dot(h, w2_ref[0], preferred_element_type=jnp.float32)
    out_ref[...] = out * ws_ref[:, 0:1]


def _run_ffn(counts, ein, w1, w2, wslot, interpret=False):
    nblk = NROWS // C
    grid = (nblk,)
    return pl.pallas_call(
        _ffn_body,
        grid=grid,
        in_specs=[
            pl.BlockSpec(memory_space=pltpu.SMEM),
            pl.BlockSpec((C, D), lambda i: (i, 0)),
            pl.BlockSpec((1, D, F), lambda i: (jnp.minimum(i, E - 1), 0, 0)),
            pl.BlockSpec((1, F, D), lambda i: (jnp.minimum(i, E - 1), 0, 0)),
            pl.BlockSpec((C, 128), lambda i: (i, 0)),
        ],
        out_specs=pl.BlockSpec((C, D), lambda i: (i, 0)),
        out_shape=jax.ShapeDtypeStruct((NROWS, D), jnp.float32),
        interpret=interpret,
    )(counts, ein, w1, w2, wslot)


def _combine_body(outw_hbm, s_hbm, y_hbm, idx_v, bufa, bufb, sem):
    cid = lax.axis_index("c")
    sid = lax.axis_index("s")
    w = sid * 2 + cid
    t0 = w * (T // NW)
    pltpu.sync_copy(s_hbm.at[w], idx_v)
    for u in range(2):
        pltpu.async_copy(outw_hbm.at[idx_v.at[0, pl.ds(u * DCH, DCH)]],
                         bufa, sem).wait()
        pltpu.async_copy(outw_hbm.at[idx_v.at[1, pl.ds(u * DCH, DCH)]],
                         bufb, sem).wait()

        def rbody(r):
            for cc in range(D // 16):
                sl = pl.ds(cc * 16, 16)
                bufa[r, sl] = bufa[r, sl] + bufb[r, sl]
        pl.loop(0, DCH)(rbody)
        pltpu.sync_copy(bufa, y_hbm.at[pl.ds(t0 + u * DCH, DCH)])


def _run_combine(outw, s_d):
    mesh = plsc.VectorSubcoreMesh(core_axis_name="c", subcore_axis_name="s")
    kern = functools.partial(
        pl.kernel,
        out_type=jax.ShapeDtypeStruct((T, D), jnp.float32),
        mesh=mesh,
        scratch_types=[
            pltpu.VMEM((2, T // NW), jnp.int32),
            pltpu.VMEM((DCH, D), jnp.float32),
            pltpu.VMEM((DCH, D), jnp.float32),
            pltpu.SemaphoreType.DMA,
        ],
    )
    return kern(_combine_body)(outw, s_d)


def kernel(x, w_gate, w1, w2):
    wg_pad = jnp.pad(w_gate, ((0, 0), (0, LANES - E)))
    d0, d1, wrow, counts, aux = _run_router(x, wg_pad)

    dk = jnp.stack([d0[:, 0], d1[:, 0]])
    d_b = dk.reshape(2, 16, 2, CHUNK).reshape(NW, 2, CHUNK)
    w_bb = wrow.reshape(2, 16, 2, CHUNK, 128).reshape(NW, 2, CHUNK, 128)
    ein, wslot = _run_dispatch(x, d_b, w_bb)

    outw = _run_ffn(counts, ein, w1.astype(jnp.bfloat16),
                    w2.astype(jnp.bfloat16), wslot)

    s_d = dk.reshape(2, NW, T // NW).transpose(1, 0, 2)
    y = _run_combine(outw, s_d)
    return y, aux[0, 0]


# speedup vs baseline: 1.3734x; 1.3734x over previous
import functools

import jax
import jax.numpy as jnp
from jax import lax
from jax.experimental import pallas as pl
from jax.experimental.pallas import tpu as pltpu
from jax.experimental.pallas import tpu_sc as plsc

E = 8
K = 2
D = 1024
F = 2048
T = 2048
C = 640
NROWS = (E + 1) * C
DUMP = E * C
LANES = 128
NW = 32
CHUNK = 64
DCH = 32


def _router_body(x_ref, wg_ref, d0_ref, d1_ref, wrow_ref,
                 counts_ref, aux_ref):
    x = x_ref[...]
    wg = wg_ref[...]
    logits = jnp.dot(x, wg, preferred_element_type=jnp.float32)
    lane = lax.broadcasted_iota(jnp.int32, (T, LANES), 1).astype(jnp.float32)
    valid = lane < float(E)
    m = jnp.max(jnp.where(valid, logits, -jnp.inf), axis=1, keepdims=True)
    ex = jnp.where(valid, jnp.exp(logits - m), 0.0)
    z = jnp.sum(ex, axis=1, keepdims=True)
    probs = ex / z

    m1 = jnp.max(probs, axis=1, keepdims=True)
    is1 = jnp.logical_and(probs == m1, valid)
    i1 = jnp.min(jnp.where(is1, lane, float(LANES)), axis=1, keepdims=True)
    mask0 = (lane == i1).astype(jnp.float32)
    p2 = jnp.where(mask0 > 0, -1.0, probs)
    m2 = jnp.max(p2, axis=1, keepdims=True)
    is2 = jnp.logical_and(p2 == m2, valid)
    i2 = jnp.min(jnp.where(is2, lane, float(LANES)), axis=1, keepdims=True)
    mask1 = (lane == i2).astype(jnp.float32)

    denom = m1 + m2 + 1e-9
    g0 = m1 / denom
    g1 = m2 / denom

    B = 256
    r = lax.broadcasted_iota(jnp.int32, (B, B), 0)
    c = lax.broadcasted_iota(jnp.int32, (B, B), 1)
    ltri = (r > c).astype(jnp.float32)
    carry = jnp.zeros((1, LANES), dtype=jnp.float32)
    pos_parts = []
    for mask in (mask0, mask1):
        parts = []
        for b in range(T // B):
            mb = mask[b * B:(b + 1) * B, :]
            parts.append(jnp.dot(ltri, mb, preferred_element_type=jnp.float32)
                         + carry)
            carry = carry + jnp.sum(mb, axis=0, keepdims=True)
        pos_parts.append(jnp.concatenate(parts, axis=0))
    pos0, pos1 = pos_parts
    total = carry

    p0 = jnp.sum(pos0 * mask0, axis=1, keepdims=True)
    p1 = jnp.sum(pos1 * mask1, axis=1, keepdims=True)
    keep0 = p0 < float(C)
    keep1 = p1 < float(C)
    d0 = jnp.where(keep0, i1 * float(C) + p0, float(DUMP))
    d1 = jnp.where(keep1, i2 * float(C) + p1, float(DUMP))
    d0_ref[...] = d0.astype(jnp.int32)
    d1_ref[...] = d1.astype(jnp.int32)
    w0e = jnp.where(keep0, g0, 0.0)
    w1e = jnp.where(keep1, g1, 0.0)
    w_all = jnp.concatenate([w0e, w1e], axis=0)
    wrow_ref[...] = jnp.broadcast_to(w_all, (K * T, 128))
    counts_ref[...] = jnp.minimum(total, float(C))

    em = jnp.maximum(mask0, mask1)
    tpe = jnp.sum(em, axis=0, keepdims=True) * (1.0 / T)
    ppe = jnp.sum(probs, axis=0, keepdims=True) * (1.0 / T)
    aux_ref[...] = jnp.sum(tpe * ppe, axis=1, keepdims=True) * float(E)


def _run_router(x, wg_pad, interpret=False):
    out_shapes = (
        jax.ShapeDtypeStruct((T, 1), jnp.int32),
        jax.ShapeDtypeStruct((T, 1), jnp.int32),
        jax.ShapeDtypeStruct((K * T, 128), jnp.float32),
        jax.ShapeDtypeStruct((1, LANES), jnp.float32),
        jax.ShapeDtypeStruct((1, 1), jnp.float32),
    )
    return pl.pallas_call(
        _router_body,
        out_shape=out_shapes,
        interpret=interpret,
    )(x, wg_pad)


def _dispatch_body(x_hbm, d_hbm, wbb_hbm, ein_hbm, wslot_hbm,
                   idx_v, rows_v, wrows_v, sem, sem2):
    cid = lax.axis_index("c")
    sid = lax.axis_index("s")
    w = sid * 2 + cid
    t0 = (w % 16) * (T // 16)
    pltpu.sync_copy(d_hbm.at[w], idx_v)
    pltpu.sync_copy(wbb_hbm.at[w], wrows_v)
    for j in range(2):
        pltpu.sync_copy(x_hbm.at[pl.ds(t0 + j * CHUNK, CHUNK)], rows_v)
        cp1 = pltpu.async_copy(rows_v, ein_hbm.at[idx_v.at[j]], sem)
        cp2 = pltpu.async_copy(wrows_v.at[j], wslot_hbm.at[idx_v.at[j]], sem2)
        cp1.wait()
        cp2.wait()


def _run_dispatch(x, d_b, w_bb):
    mesh = plsc.VectorSubcoreMesh(core_axis_name="c", subcore_axis_name="s")
    kern = functools.partial(
        pl.kernel,
        out_type=(
            jax.ShapeDtypeStruct((NROWS, D), jnp.float32),
            jax.ShapeDtypeStruct((NROWS, 128), jnp.float32),
        ),
        mesh=mesh,
        scratch_types=[
            pltpu.VMEM((2, CHUNK), jnp.int32),
            pltpu.VMEM((CHUNK, D), jnp.float32),
            pltpu.VMEM((2, CHUNK, 128), jnp.float32),
            pltpu.SemaphoreType.DMA,
            pltpu.SemaphoreType.DMA,
        ],
    )
    return kern(_dispatch_body)(x, d_b, w_bb)


def _ffn_body(counts_ref, xin_ref, w1_ref, w2_ref, ws_ref, out_ref):
    e = pl.program_id(0)
    cnt = counts_ref[0, e]
    row = lax.broadcasted_iota(jnp.int32, (C, 1), 0).astype(jnp.float32)
    x = jnp.where(row < cnt, xin_ref[...], 0.0).astype(jnp.bfloat16)
    h = jnp.dot(x, w1_ref[0].astype(jnp.bfloat16),
                preferred_element_type=jnp.float32)
    h = jax.nn.gelu(h, approximate=True).astype(jnp.bfloat16)
    out = jnp.dot(h, w2_ref[0].astype(jnp.bfloat16),
                  preferred_element_type=jnp.float32)
    out_ref[...] = out * ws_ref[:, 0:1]


def _run_ffn(counts, ein, w1, w2, wslot, interpret=False):
    nblk = NROWS // C
    grid = (nblk,)
    return pl.pallas_call(
        _ffn_body,
        grid=grid,
        in_specs=[
            pl.BlockSpec(memory_space=pltpu.SMEM),
            pl.BlockSpec((C, D), lambda i: (i, 0)),
            pl.BlockSpec((1, D, F), lambda i: (jnp.minimum(i, E - 1), 0, 0)),
            pl.BlockSpec((1, F, D), lambda i: (jnp.minimum(i, E - 1), 0, 0)),
            pl.BlockSpec((C, 128), lambda i: (i, 0)),
        ],
        out_specs=pl.BlockSpec((C, D), lambda i: (i, 0)),
        out_shape=jax.ShapeDtypeStruct((NROWS, D), jnp.float32),
        interpret=interpret,
    )(counts, ein, w1, w2, wslot)


def _combine_body(outw_hbm, s_hbm, y_hbm, idx_v, bufa, bufb, sem):
    cid = lax.axis_index("c")
    sid = lax.axis_index("s")
    w = sid * 2 + cid
    t0 = w * (T // NW)
    pltpu.sync_copy(s_hbm.at[w], idx_v)
    for u in range(2):
        pltpu.async_copy(outw_hbm.at[idx_v.at[0, pl.ds(u * DCH, DCH)]],
                         bufa, sem).wait()
        pltpu.async_copy(outw_hbm.at[idx_v.at[1, pl.ds(u * DCH, DCH)]],
                         bufb, sem).wait()

        def rbody(r):
            for cc in range(D // 16):
                sl = pl.ds(cc * 16, 16)
                bufa[r, sl] = bufa[r, sl] + bufb[r, sl]
        pl.loop(0, DCH)(rbody)
        pltpu.sync_copy(bufa, y_hbm.at[pl.ds(t0 + u * DCH, DCH)])


def _run_combine(outw, s_d):
    mesh = plsc.VectorSubcoreMesh(core_axis_name="c", subcore_axis_name="s")
    kern = functools.partial(
        pl.kernel,
        out_type=jax.ShapeDtypeStruct((T, D), jnp.float32),
        mesh=mesh,
        scratch_types=[
            pltpu.VMEM((2, T // NW), jnp.int32),
            pltpu.VMEM((DCH, D), jnp.float32),
            pltpu.VMEM((DCH, D), jnp.float32),
            pltpu.SemaphoreType.DMA,
        ],
    )
    return kern(_combine_body)(outw, s_d)


def kernel(x, w_gate, w1, w2):
    wg_pad = jnp.pad(w_gate, ((0, 0), (0, LANES - E)))
    d0, d1, wrow, counts, aux = _run_router(x, wg_pad)

    dk = jnp.stack([d0[:, 0], d1[:, 0]])
    d_b = dk.reshape(2, 16, 2, CHUNK).reshape(NW, 2, CHUNK)
    w_bb = wrow.reshape(2, 16, 2, CHUNK, 128).reshape(NW, 2, CHUNK, 128)
    ein, wslot = _run_dispatch(x, d_b, w_bb)

    outw = _run_ffn(counts, ein, w1, w2, wslot)

    s_d = dk.reshape(2, NW, T // NW).transpose(1, 0, 2)
    y = _run_combine(outw, s_d)
    return y, aux[0, 0]


# speedup vs baseline: 1.4778x; 1.0760x over previous
import functools

import jax
import jax.numpy as jnp
from jax import lax
from jax.experimental import pallas as pl
from jax.experimental.pallas import tpu as pltpu
from jax.experimental.pallas import tpu_sc as plsc

E = 8
K = 2
D = 1024
F = 2048
T = 2048
C = 640
NROWS = (E + 1) * C
DUMP = E * C
LANES = 128
NW = 32
BCH = 32
DCH = 16


def _router_body(x_ref, wg_ref, d0_ref, d1_ref, wrow_ref,
                 counts_ref, aux_ref):
    x = x_ref[...]
    wg = wg_ref[...]
    logits = jnp.dot(x, wg, preferred_element_type=jnp.float32)
    lane = lax.broadcasted_iota(jnp.int32, (T, LANES), 1).astype(jnp.float32)
    valid = lane < float(E)
    m = jnp.max(jnp.where(valid, logits, -jnp.inf), axis=1, keepdims=True)
    ex = jnp.where(valid, jnp.exp(logits - m), 0.0)
    z = jnp.sum(ex, axis=1, keepdims=True)
    probs = ex / z

    m1 = jnp.max(probs, axis=1, keepdims=True)
    is1 = jnp.logical_and(probs == m1, valid)
    i1 = jnp.min(jnp.where(is1, lane, float(LANES)), axis=1, keepdims=True)
    mask0 = (lane == i1).astype(jnp.float32)
    p2 = jnp.where(mask0 > 0, -1.0, probs)
    m2 = jnp.max(p2, axis=1, keepdims=True)
    is2 = jnp.logical_and(p2 == m2, valid)
    i2 = jnp.min(jnp.where(is2, lane, float(LANES)), axis=1, keepdims=True)
    mask1 = (lane == i2).astype(jnp.float32)

    denom = m1 + m2 + 1e-9
    g0 = m1 / denom
    g1 = m2 / denom

    B = 256
    r = lax.broadcasted_iota(jnp.int32, (B, B), 0)
    c = lax.broadcasted_iota(jnp.int32, (B, B), 1)
    ltri = (r > c).astype(jnp.float32)
    carry = jnp.zeros((1, LANES), dtype=jnp.float32)
    pos_parts = []
    for mask in (mask0, mask1):
        parts = []
        for b in range(T // B):
            mb = mask[b * B:(b + 1) * B, :]
            parts.append(jnp.dot(ltri, mb, preferred_element_type=jnp.float32)
                         + carry)
            carry = carry + jnp.sum(mb, axis=0, keepdims=True)
        pos_parts.append(jnp.concatenate(parts, axis=0))
    pos0, pos1 = pos_parts
    total = carry

    p0 = jnp.sum(pos0 * mask0, axis=1, keepdims=True)
    p1 = jnp.sum(pos1 * mask1, axis=1, keepdims=True)
    keep0 = p0 < float(C)
    keep1 = p1 < float(C)
    d0 = jnp.where(keep0, i1 * float(C) + p0, float(DUMP))
    d1 = jnp.where(keep1, i2 * float(C) + p1, float(DUMP))
    d0_ref[...] = d0.astype(jnp.int32)
    d1_ref[...] = d1.astype(jnp.int32)
    w0e = jnp.where(keep0, g0, 0.0)
    w1e = jnp.where(keep1, g1, 0.0)
    w_all = jnp.concatenate([w0e, w1e], axis=0)
    wrow_ref[...] = jnp.broadcast_to(w_all, (K * T, 128))
    counts_ref[...] = jnp.minimum(total, float(C))

    em = jnp.maximum(mask0, mask1)
    tpe = jnp.sum(em, axis=0, keepdims=True) * (1.0 / T)
    ppe = jnp.sum(probs, axis=0, keepdims=True) * (1.0 / T)
    aux_ref[...] = jnp.sum(tpe * ppe, axis=1, keepdims=True) * float(E)


def _run_router(x, wg_pad, interpret=False):
    out_shapes = (
        jax.ShapeDtypeStruct((T, 1), jnp.int32),
        jax.ShapeDtypeStruct((T, 1), jnp.int32),
        jax.ShapeDtypeStruct((K * T, 128), jnp.float32),
        jax.ShapeDtypeStruct((1, LANES), jnp.float32),
        jax.ShapeDtypeStruct((1, 1), jnp.float32),
    )
    return pl.pallas_call(
        _router_body,
        out_shape=out_shapes,
        interpret=interpret,
    )(x, wg_pad)


def _dispatch_body(x_hbm, d_hbm, wbb_hbm, ein_hbm, wslot_hbm,
                   idx_v, b0, b1, b2, wrows_v,
                   sa0, sa1, sa2, sb0, sb1, sb2, semw):
    cid = lax.axis_index("c")
    sid = lax.axis_index("s")
    w = sid * 2 + cid
    t0 = (w % 16) * (T // 16)
    bufs = (b0, b1, b2)
    sa = (sa0, sa1, sa2)
    sb = (sb0, sb1, sb2)
    stages = [pltpu.async_copy(x_hbm.at[pl.ds(t0 + j * BCH, BCH)],
                               bufs[j], sa[j]) for j in range(3)]
    pltpu.sync_copy(d_hbm.at[w], idx_v)
    pltpu.sync_copy(wbb_hbm.at[w], wrows_v)
    scat = [None] * 4
    wscat = []
    for j in range(4):
        if j == 3:
            scat[0].wait()
            stages.append(pltpu.async_copy(
                x_hbm.at[pl.ds(t0 + 3 * BCH, BCH)], b0, sa0))
        stages[j].wait()
        scat[j] = pltpu.async_copy(bufs[j % 3], ein_hbm.at[idx_v.at[j]],
                                   sb[j % 3])
        wscat.append(pltpu.async_copy(wrows_v.at[j],
                                      wslot_hbm.at[idx_v.at[j]], semw))
    for j in (1, 2, 3):
        scat[j].wait()
    for cp in wscat:
        cp.wait()


def _run_dispatch(x, d_b, w_bb):
    mesh = plsc.VectorSubcoreMesh(core_axis_name="c", subcore_axis_name="s")
    kern = functools.partial(
        pl.kernel,
        out_type=(
            jax.ShapeDtypeStruct((NROWS, D), jnp.float32),
            jax.ShapeDtypeStruct((NROWS, 128), jnp.float32),
        ),
        mesh=mesh,
        scratch_types=[
            pltpu.VMEM((4, BCH), jnp.int32),
            pltpu.VMEM((BCH, D), jnp.float32),
            pltpu.VMEM((BCH, D), jnp.float32),
            pltpu.VMEM((BCH, D), jnp.float32),
            pltpu.VMEM((4, BCH, 128), jnp.float32),
            pltpu.SemaphoreType.DMA,
            pltpu.SemaphoreType.DMA,
            pltpu.SemaphoreType.DMA,
            pltpu.SemaphoreType.DMA,
            pltpu.SemaphoreType.DMA,
            pltpu.SemaphoreType.DMA,
            pltpu.SemaphoreType.DMA,
        ],
    )
    return kern(_dispatch_body)(x, d_b, w_bb)


def _ffn_body(counts_ref, xin_ref, w1_ref, w2_ref, ws_ref, out_ref):
    e = pl.program_id(0)

    @pl.when(e < E)
    def _():
        cnt = counts_ref[0, e]
        row = lax.broadcasted_iota(jnp.int32, (C, 1), 0).astype(jnp.float32)
        x = jnp.where(row < cnt, xin_ref[...], 0.0).astype(jnp.bfloat16)
        h = jnp.dot(x, w1_ref[0].astype(jnp.bfloat16),
                    preferred_element_type=jnp.float32)
        h = jax.nn.gelu(h, approximate=True).astype(jnp.bfloat16)
        out = jnp.dot(h, w2_ref[0].astype(jnp.bfloat16),
                      preferred_element_type=jnp.float32)
        out_ref[...] = out * ws_ref[:, 0:1]

    @pl.when(e >= E)
    def _():
        out_ref[...] = jnp.zeros((C, D), jnp.float32)


def _run_ffn(counts, ein, w1, w2, wslot, interpret=False):
    nblk = NROWS // C
    grid = (nblk,)
    return pl.pallas_call(
        _ffn_body,
        grid=grid,
        in_specs=[
            pl.BlockSpec(memory_space=pltpu.SMEM),
            pl.BlockSpec((C, D), lambda i: (i, 0)),
            pl.BlockSpec((1, D, F), lambda i: (jnp.minimum(i, E - 1), 0, 0)),
            pl.BlockSpec((1, F, D), lambda i: (jnp.minimum(i, E - 1), 0, 0)),
            pl.BlockSpec((C, 128), lambda i: (i, 0)),
        ],
        out_specs=pl.BlockSpec((C, D), lambda i: (i, 0)),
        out_shape=jax.ShapeDtypeStruct((NROWS, D), jnp.float32),
        interpret=interpret,
    )(counts, ein, w1, w2, wslot)


def _combine_body(outw_hbm, s_hbm, y_hbm, idx_v,
                  a0, b0_, a1, b1_, sga0, sgb0, sga1, sgb1, swo0, swo1):
    cid = lax.axis_index("c")
    sid = lax.axis_index("s")
    w = sid * 2 + cid
    t0 = w * (T // NW)
    nsub = (T // NW) // DCH
    pltpu.sync_copy(s_hbm.at[w], idx_v)
    pa = (a0, a1)
    pb = (b0_, b1_)
    sga = (sga0, sga1)
    sgb = (sgb0, sgb1)
    swo = (swo0, swo1)

    def gathers(u):
        p = u % 2
        ga = pltpu.async_copy(
            outw_hbm.at[idx_v.at[0, pl.ds(u * DCH, DCH)]], pa[p], sga[p])
        gb = pltpu.async_copy(
            outw_hbm.at[idx_v.at[1, pl.ds(u * DCH, DCH)]], pb[p], sgb[p])
        return ga, gb

    g = gathers(0)
    wo = [None, None]
    for u in range(nsub):
        p = u % 2
        if u + 1 < nsub:
            if wo[(u + 1) % 2] is not None:
                wo[(u + 1) % 2].wait()
            gnext = gathers(u + 1)
        g[0].wait()
        g[1].wait()

        def rbody(r):
            for cc in range(D // 16):
                sl = pl.ds(cc * 16, 16)
                pa[p][r, sl] = pa[p][r, sl] + pb[p][r, sl]
        pl.loop(0, DCH)(rbody)
        wo[p] = pltpu.async_copy(pa[p], y_hbm.at[pl.ds(t0 + u * DCH, DCH)],
                                 swo[p])
        if u + 1 < nsub:
            g = gnext
    wo[0].wait()
    wo[1].wait()


def _run_combine(outw, s_d):
    mesh = plsc.VectorSubcoreMesh(core_axis_name="c", subcore_axis_name="s")
    kern = functools.partial(
        pl.kernel,
        out_type=jax.ShapeDtypeStruct((T, D), jnp.float32),
        mesh=mesh,
        scratch_types=[
            pltpu.VMEM((2, T // NW), jnp.int32),
            pltpu.VMEM((DCH, D), jnp.float32),
            pltpu.VMEM((DCH, D), jnp.float32),
            pltpu.VMEM((DCH, D), jnp.float32),
            pltpu.VMEM((DCH, D), jnp.float32),
            pltpu.SemaphoreType.DMA,
            pltpu.SemaphoreType.DMA,
            pltpu.SemaphoreType.DMA,
            pltpu.SemaphoreType.DMA,
            pltpu.SemaphoreType.DMA,
            pltpu.SemaphoreType.DMA,
        ],
    )
    return kern(_combine_body)(outw, s_d)


def kernel(x, w_gate, w1, w2):
    wg_pad = jnp.pad(w_gate, ((0, 0), (0, LANES - E)))
    d0, d1, wrow, counts, aux = _run_router(x, wg_pad)

    dk = jnp.stack([d0[:, 0], d1[:, 0]])
    d_b = dk.reshape(2, 16, 4, BCH).reshape(NW, 4, BCH)
    w_bb = wrow.reshape(2, 16, 4, BCH, 128).reshape(NW, 4, BCH, 128)
    ein, wslot = _run_dispatch(x, d_b, w_bb)

    outw = _run_ffn(counts, ein, w1, w2, wslot)

    s_d = dk.reshape(2, NW, T // NW).transpose(1, 0, 2)
    y = _run_combine(outw, s_d)
    return y, aux[0, 0]


# speedup vs baseline: 1.5548x; 1.0521x over previous
import functools

import jax
import jax.numpy as jnp
from jax import lax
from jax.experimental import pallas as pl
from jax.experimental.pallas import tpu as pltpu
from jax.experimental.pallas import tpu_sc as plsc

E = 8
K = 2
D = 1024
F = 2048
T = 2048
C = 640
NROWS = (E + 1) * C
DUMP = E * C
LANES = 128
NW = 32
BCH = 32
DCH = 16


def _router_body(x_ref, wg_ref, d0_ref, d1_ref, wrow_ref,
                 counts_ref, aux_ref, x16_ref):
    x = x_ref[...]
    rt = x.astype(jnp.bfloat16).astype(jnp.float32)
    bits = lax.bitcast_convert_type(rt, jnp.uint32)
    lo = bits[:, :D // 2] >> 16
    hi = bits[:, D // 2:] & jnp.uint32(0xFFFF0000)
    x16_ref[...] = lax.bitcast_convert_type(lo | hi, jnp.int32)
    wg = wg_ref[...]
    logits = jnp.dot(x, wg, preferred_element_type=jnp.float32)
    lane = lax.broadcasted_iota(jnp.int32, (T, LANES), 1).astype(jnp.float32)
    valid = lane < float(E)
    m = jnp.max(jnp.where(valid, logits, -jnp.inf), axis=1, keepdims=True)
    ex = jnp.where(valid, jnp.exp(logits - m), 0.0)
    z = jnp.sum(ex, axis=1, keepdims=True)
    probs = ex / z

    m1 = jnp.max(probs, axis=1, keepdims=True)
    is1 = jnp.logical_and(probs == m1, valid)
    i1 = jnp.min(jnp.where(is1, lane, float(LANES)), axis=1, keepdims=True)
    mask0 = (lane == i1).astype(jnp.float32)
    p2 = jnp.where(mask0 > 0, -1.0, probs)
    m2 = jnp.max(p2, axis=1, keepdims=True)
    is2 = jnp.logical_and(p2 == m2, valid)
    i2 = jnp.min(jnp.where(is2, lane, float(LANES)), axis=1, keepdims=True)
    mask1 = (lane == i2).astype(jnp.float32)

    denom = m1 + m2 + 1e-9
    g0 = m1 / denom
    g1 = m2 / denom

    B = 256
    r = lax.broadcasted_iota(jnp.int32, (B, B), 0)
    c = lax.broadcasted_iota(jnp.int32, (B, B), 1)
    ltri = (r > c).astype(jnp.float32)
    carry = jnp.zeros((1, LANES), dtype=jnp.float32)
    pos_parts = []
    for mask in (mask0, mask1):
        parts = []
        for b in range(T // B):
            mb = mask[b * B:(b + 1) * B, :]
            parts.append(jnp.dot(ltri, mb, preferred_element_type=jnp.float32)
                         + carry)
            carry = carry + jnp.sum(mb, axis=0, keepdims=True)
        pos_parts.append(jnp.concatenate(parts, axis=0))
    pos0, pos1 = pos_parts
    total = carry

    p0 = jnp.sum(pos0 * mask0, axis=1, keepdims=True)
    p1 = jnp.sum(pos1 * mask1, axis=1, keepdims=True)
    keep0 = p0 < float(C)
    keep1 = p1 < float(C)
    d0 = jnp.where(keep0, i1 * float(C) + p0, float(DUMP))
    d1 = jnp.where(keep1, i2 * float(C) + p1, float(DUMP))
    d0_ref[...] = d0.astype(jnp.int32)
    d1_ref[...] = d1.astype(jnp.int32)
    w0e = jnp.where(keep0, g0, 0.0)
    w1e = jnp.where(keep1, g1, 0.0)
    w_all = jnp.concatenate([w0e, w1e], axis=0)
    wrow_ref[...] = jnp.broadcast_to(w_all, (K * T, 128))
    counts_ref[...] = jnp.minimum(total, float(C))

    em = jnp.maximum(mask0, mask1)
    tpe = jnp.sum(em, axis=0, keepdims=True) * (1.0 / T)
    ppe = jnp.sum(probs, axis=0, keepdims=True) * (1.0 / T)
    aux_ref[...] = jnp.sum(tpe * ppe, axis=1, keepdims=True) * float(E)


def _run_router(x, wg_pad, interpret=False):
    out_shapes = (
        jax.ShapeDtypeStruct((T, 1), jnp.int32),
        jax.ShapeDtypeStruct((T, 1), jnp.int32),
        jax.ShapeDtypeStruct((K * T, 128), jnp.float32),
        jax.ShapeDtypeStruct((1, LANES), jnp.float32),
        jax.ShapeDtypeStruct((1, 1), jnp.float32),
        jax.ShapeDtypeStruct((T, D // 2), jnp.int32),
    )
    return pl.pallas_call(
        _router_body,
        out_shape=out_shapes,
        interpret=interpret,
    )(x, wg_pad)


def _dispatch_body(x_hbm, d_hbm, wbb_hbm, ein_hbm, wslot_hbm,
                   idx_v, b0, b1, b2, wrows_v,
                   sa0, sa1, sa2, sb0, sb1, sb2, semw):
    cid = lax.axis_index("c")
    sid = lax.axis_index("s")
    w = sid * 2 + cid
    t0 = (w % 16) * (T // 16)
    bufs = (b0, b1, b2)
    sa = (sa0, sa1, sa2)
    sb = (sb0, sb1, sb2)
    stages = [pltpu.async_copy(x_hbm.at[pl.ds(t0 + j * BCH, BCH)],
                               bufs[j], sa[j]) for j in range(3)]
    pltpu.sync_copy(d_hbm.at[w], idx_v)
    pltpu.sync_copy(wbb_hbm.at[w], wrows_v)
    scat = [None] * 4
    wscat = []
    for j in range(4):
        if j == 3:
            scat[0].wait()
            stages.append(pltpu.async_copy(
                x_hbm.at[pl.ds(t0 + 3 * BCH, BCH)], b0, sa0))
        stages[j].wait()
        scat[j] = pltpu.async_copy(bufs[j % 3], ein_hbm.at[idx_v.at[j]],
                                   sb[j % 3])
        wscat.append(pltpu.async_copy(wrows_v.at[j],
                                      wslot_hbm.at[idx_v.at[j]], semw))
    for j in (1, 2, 3):
        scat[j].wait()
    for cp in wscat:
        cp.wait()


def _run_dispatch(x, d_b, w_bb):
    mesh = plsc.VectorSubcoreMesh(core_axis_name="c", subcore_axis_name="s")
    kern = functools.partial(
        pl.kernel,
        out_type=(
            jax.ShapeDtypeStruct((NROWS, D // 2), jnp.int32),
            jax.ShapeDtypeStruct((NROWS, 128), jnp.float32),
        ),
        mesh=mesh,
        scratch_types=[
            pltpu.VMEM((4, BCH), jnp.int32),
            pltpu.VMEM((BCH, D // 2), jnp.int32),
            pltpu.VMEM((BCH, D // 2), jnp.int32),
            pltpu.VMEM((BCH, D // 2), jnp.int32),
            pltpu.VMEM((4, BCH, 128), jnp.float32),
            pltpu.SemaphoreType.DMA,
            pltpu.SemaphoreType.DMA,
            pltpu.SemaphoreType.DMA,
            pltpu.SemaphoreType.DMA,
            pltpu.SemaphoreType.DMA,
            pltpu.SemaphoreType.DMA,
            pltpu.SemaphoreType.DMA,
        ],
    )
    return kern(_dispatch_body)(x, d_b, w_bb)


def _ffn_body(counts_ref, xin_ref, w1_ref, w2_ref, ws_ref, out_ref):
    e = pl.program_id(0)

    @pl.when(e < E)
    def _():
        cnt = counts_ref[0, e]
        row = lax.broadcasted_iota(jnp.int32, (C, 1), 0).astype(jnp.float32)
        u = lax.bitcast_convert_type(xin_ref[...], jnp.uint32)
        xlo = lax.bitcast_convert_type(u << 16, jnp.float32)
        xhi = lax.bitcast_convert_type(u & jnp.uint32(0xFFFF0000),
                                       jnp.float32)
        xp = jnp.concatenate([xlo, xhi], axis=1).astype(jnp.bfloat16)
        x = jnp.where(row < cnt, xp, jnp.zeros((C, D), jnp.bfloat16))
        h = jnp.dot(x, w1_ref[0].astype(jnp.bfloat16),
                    preferred_element_type=jnp.float32)
        h = jax.nn.gelu(h, approximate=True).astype(jnp.bfloat16)
        out = jnp.dot(h, w2_ref[0].astype(jnp.bfloat16),
                      preferred_element_type=jnp.float32)
        out_ref[...] = out * ws_ref[:, 0:1]

    @pl.when(e >= E)
    def _():
        out_ref[...] = jnp.zeros((C, D), jnp.float32)


def _run_ffn(counts, ein, w1, w2, wslot, interpret=False):
    nblk = NROWS // C
    grid = (nblk,)
    return pl.pallas_call(
        _ffn_body,
        grid=grid,
        in_specs=[
            pl.BlockSpec(memory_space=pltpu.SMEM),
            pl.BlockSpec((C, D // 2), lambda i: (i, 0)),
            pl.BlockSpec((1, D, F), lambda i: (jnp.minimum(i, E - 1), 0, 0)),
            pl.BlockSpec((1, F, D), lambda i: (jnp.minimum(i, E - 1), 0, 0)),
            pl.BlockSpec((C, 128), lambda i: (i, 0)),
        ],
        out_specs=pl.BlockSpec((C, D), lambda i: (i, 0)),
        out_shape=jax.ShapeDtypeStruct((NROWS, D), jnp.float32),
        interpret=interpret,
    )(counts, ein, w1, w2, wslot)


def _combine_body(outw_hbm, s_hbm, y_hbm, idx_v,
                  a0, b0_, a1, b1_, sga0, sgb0, sga1, sgb1, swo0, swo1):
    cid = lax.axis_index("c")
    sid = lax.axis_index("s")
    w = sid * 2 + cid
    t0 = w * (T // NW)
    nsub = (T // NW) // DCH
    pltpu.sync_copy(s_hbm.at[w], idx_v)
    pa = (a0, a1)
    pb = (b0_, b1_)
    sga = (sga0, sga1)
    sgb = (sgb0, sgb1)
    swo = (swo0, swo1)

    def gathers(u):
        p = u % 2
        ga = pltpu.async_copy(
            outw_hbm.at[idx_v.at[0, pl.ds(u * DCH, DCH)]], pa[p], sga[p])
        gb = pltpu.async_copy(
            outw_hbm.at[idx_v.at[1, pl.ds(u * DCH, DCH)]], pb[p], sgb[p])
        return ga, gb

    g = gathers(0)
    wo = [None, None]
    for u in range(nsub):
        p = u % 2
        if u + 1 < nsub:
            if wo[(u + 1) % 2] is not None:
                wo[(u + 1) % 2].wait()
            gnext = gathers(u + 1)
        g[0].wait()
        g[1].wait()

        def rbody(r):
            for cc in range(D // 16):
                sl = pl.ds(cc * 16, 16)
                pa[p][r, sl] = pa[p][r, sl] + pb[p][r, sl]
        pl.loop(0, DCH)(rbody)
        wo[p] = pltpu.async_copy(pa[p], y_hbm.at[pl.ds(t0 + u * DCH, DCH)],
                                 swo[p])
        if u + 1 < nsub:
            g = gnext
    wo[0].wait()
    wo[1].wait()


def _run_combine(outw, s_d):
    mesh = plsc.VectorSubcoreMesh(core_axis_name="c", subcore_axis_name="s")
    kern = functools.partial(
        pl.kernel,
        out_type=jax.ShapeDtypeStruct((T, D), jnp.float32),
        mesh=mesh,
        scratch_types=[
            pltpu.VMEM((2, T // NW), jnp.int32),
            pltpu.VMEM((DCH, D), jnp.float32),
            pltpu.VMEM((DCH, D), jnp.float32),
            pltpu.VMEM((DCH, D), jnp.float32),
            pltpu.VMEM((DCH, D), jnp.float32),
            pltpu.SemaphoreType.DMA,
            pltpu.SemaphoreType.DMA,
            pltpu.SemaphoreType.DMA,
            pltpu.SemaphoreType.DMA,
            pltpu.SemaphoreType.DMA,
            pltpu.SemaphoreType.DMA,
        ],
    )
    return kern(_combine_body)(outw, s_d)


def kernel(x, w_gate, w1, w2):
    wg_pad = jnp.pad(w_gate, ((0, 0), (0, LANES - E)))
    d0, d1, wrow, counts, aux, x16 = _run_router(x, wg_pad)

    dk = jnp.stack([d0[:, 0], d1[:, 0]])
    d_b = dk.reshape(2, 16, 4, BCH).reshape(NW, 4, BCH)
    w_bb = wrow.reshape(2, 16, 4, BCH, 128).reshape(NW, 4, BCH, 128)
    ein, wslot = _run_dispatch(x16, d_b, w_bb)

    outw = _run_ffn(counts, ein, w1, w2, wslot)

    s_d = dk.reshape(2, NW, T // NW).transpose(1, 0, 2)
    y = _run_combine(outw, s_d)
    return y, aux[0, 0]


# speedup vs baseline: 1.5553x; 1.0003x over previous
import functools

import jax
import jax.numpy as jnp
from jax import lax
from jax.experimental import pallas as pl
from jax.experimental.pallas import tpu as pltpu
from jax.experimental.pallas import tpu_sc as plsc

E = 8
K = 2
D = 1024
F = 2048
T = 2048
C = 640
NROWS = (E + 1) * C
DUMP = E * C
LANES = 128
NW = 32
BCH = 32
DCH = 16


def _router_body(x_ref, wg_ref, d0_ref, d1_ref, wrow_ref,
                 counts_ref, aux_ref, x16_ref):
    x = x_ref[...]
    rt = x.astype(jnp.bfloat16).astype(jnp.float32)
    bits = lax.bitcast_convert_type(rt, jnp.uint32)
    lo = bits[:, :D // 2] >> 16
    hi = bits[:, D // 2:] & jnp.uint32(0xFFFF0000)
    x16_ref[...] = lax.bitcast_convert_type(lo | hi, jnp.int32)
    wg = wg_ref[...]
    logits = jnp.dot(x, wg, preferred_element_type=jnp.float32)
    lane = lax.broadcasted_iota(jnp.int32, (T, E), 1).astype(jnp.float32)
    m = jnp.max(logits, axis=1, keepdims=True)
    ex = jnp.exp(logits - m)
    z = jnp.sum(ex, axis=1, keepdims=True)
    probs = ex / z

    m1 = jnp.max(probs, axis=1, keepdims=True)
    is1 = probs == m1
    i1 = jnp.min(jnp.where(is1, lane, float(E)), axis=1, keepdims=True)
    mask0 = (lane == i1).astype(jnp.float32)
    p2 = jnp.where(mask0 > 0, -1.0, probs)
    m2 = jnp.max(p2, axis=1, keepdims=True)
    is2 = p2 == m2
    i2 = jnp.min(jnp.where(is2, lane, float(E)), axis=1, keepdims=True)
    mask1 = (lane == i2).astype(jnp.float32)

    denom = m1 + m2 + 1e-9
    g0 = m1 / denom
    g1 = m2 / denom

    B = 256
    r = lax.broadcasted_iota(jnp.int32, (B, B), 0)
    c = lax.broadcasted_iota(jnp.int32, (B, B), 1)
    ltri = (r > c).astype(jnp.float32)
    carry = jnp.zeros((1, E), dtype=jnp.float32)
    pos_parts = []
    for mask in (mask0, mask1):
        parts = []
        for b in range(T // B):
            mb = mask[b * B:(b + 1) * B, :]
            parts.append(jnp.dot(ltri, mb, preferred_element_type=jnp.float32)
                         + carry)
            carry = carry + jnp.sum(mb, axis=0, keepdims=True)
        pos_parts.append(jnp.concatenate(parts, axis=0))
    pos0, pos1 = pos_parts
    total = carry

    p0 = jnp.sum(pos0 * mask0, axis=1, keepdims=True)
    p1 = jnp.sum(pos1 * mask1, axis=1, keepdims=True)
    keep0 = p0 < float(C)
    keep1 = p1 < float(C)
    d0 = jnp.where(keep0, i1 * float(C) + p0, float(DUMP))
    d1 = jnp.where(keep1, i2 * float(C) + p1, float(DUMP))
    d0_ref[...] = d0.astype(jnp.int32)
    d1_ref[...] = d1.astype(jnp.int32)
    w0e = jnp.where(keep0, g0, 0.0)
    w1e = jnp.where(keep1, g1, 0.0)
    w_all = jnp.concatenate([w0e, w1e], axis=0)
    wrow_ref[...] = jnp.broadcast_to(w_all, (K * T, 128))
    counts_ref[...] = jnp.concatenate(
        [jnp.minimum(total, float(C)),
         jnp.zeros((1, LANES - E), jnp.float32)], axis=1)

    em = jnp.maximum(mask0, mask1)
    tpe = jnp.sum(em, axis=0, keepdims=True) * (1.0 / T)
    ppe = jnp.sum(probs, axis=0, keepdims=True) * (1.0 / T)
    aux_ref[...] = jnp.sum(tpe * ppe, axis=1, keepdims=True) * float(E)


def _run_router(x, wg, interpret=False):
    out_shapes = (
        jax.ShapeDtypeStruct((T, 1), jnp.int32),
        jax.ShapeDtypeStruct((T, 1), jnp.int32),
        jax.ShapeDtypeStruct((K * T, 128), jnp.float32),
        jax.ShapeDtypeStruct((1, LANES), jnp.float32),
        jax.ShapeDtypeStruct((1, 1), jnp.float32),
        jax.ShapeDtypeStruct((T, D // 2), jnp.int32),
    )
    return pl.pallas_call(
        _router_body,
        out_shape=out_shapes,
        interpret=interpret,
    )(x, wg)


def _dispatch_body(x_hbm, d_hbm, wbb_hbm, ein_hbm, wslot_hbm,
                   idx_v, b0, b1, b2, wrows_v,
                   sa0, sa1, sa2, sb0, sb1, sb2, semw):
    cid = lax.axis_index("c")
    sid = lax.axis_index("s")
    w = sid * 2 + cid
    t0 = (w % 16) * (T // 16)
    bufs = (b0, b1, b2)
    sa = (sa0, sa1, sa2)
    sb = (sb0, sb1, sb2)
    stages = [pltpu.async_copy(x_hbm.at[pl.ds(t0 + j * BCH, BCH)],
                               bufs[j], sa[j]) for j in range(3)]
    pltpu.sync_copy(d_hbm.at[w], idx_v)
    pltpu.sync_copy(wbb_hbm.at[w], wrows_v)
    scat = [None] * 4
    wscat = []
    for j in range(4):
        if j == 3:
            scat[0].wait()
            stages.append(pltpu.async_copy(
                x_hbm.at[pl.ds(t0 + 3 * BCH, BCH)], b0, sa0))
        stages[j].wait()
        scat[j] = pltpu.async_copy(bufs[j % 3], ein_hbm.at[idx_v.at[j]],
                                   sb[j % 3])
        wscat.append(pltpu.async_copy(wrows_v.at[j],
                                      wslot_hbm.at[idx_v.at[j]], semw))
    for j in (1, 2, 3):
        scat[j].wait()
    for cp in wscat:
        cp.wait()


def _run_dispatch(x, d_b, w_bb):
    mesh = plsc.VectorSubcoreMesh(core_axis_name="c", subcore_axis_name="s")
    kern = functools.partial(
        pl.kernel,
        out_type=(
            jax.ShapeDtypeStruct((NROWS, D // 2), jnp.int32),
            jax.ShapeDtypeStruct((NROWS, 128), jnp.float32),
        ),
        mesh=mesh,
        scratch_types=[
            pltpu.VMEM((4, BCH), jnp.int32),
            pltpu.VMEM((BCH, D // 2), jnp.int32),
            pltpu.VMEM((BCH, D // 2), jnp.int32),
            pltpu.VMEM((BCH, D // 2), jnp.int32),
            pltpu.VMEM((4, BCH, 128), jnp.float32),
            pltpu.SemaphoreType.DMA,
            pltpu.SemaphoreType.DMA,
            pltpu.SemaphoreType.DMA,
            pltpu.SemaphoreType.DMA,
            pltpu.SemaphoreType.DMA,
            pltpu.SemaphoreType.DMA,
            pltpu.SemaphoreType.DMA,
        ],
    )
    return kern(_dispatch_body)(x, d_b, w_bb)


def _ffn_body(counts_ref, xin_ref, w1_ref, w2_ref, ws_ref, out_ref):
    e = pl.program_id(0)

    @pl.when(e < E)
    def _():
        cnt = counts_ref[0, e]
        row = lax.broadcasted_iota(jnp.int32, (C, 1), 0).astype(jnp.float32)
        u = lax.bitcast_convert_type(xin_ref[...], jnp.uint32)
        xlo = lax.bitcast_convert_type(u << 16, jnp.float32)
        xhi = lax.bitcast_convert_type(u & jnp.uint32(0xFFFF0000),
                                       jnp.float32)
        xp = jnp.concatenate([xlo, xhi], axis=1).astype(jnp.bfloat16)
        x = jnp.where(row < cnt, xp, jnp.zeros((C, D), jnp.bfloat16))
        h = jnp.dot(x, w1_ref[0].astype(jnp.bfloat16),
                    preferred_element_type=jnp.float32)
        h = jax.nn.gelu(h, approximate=True).astype(jnp.bfloat16)
        out = jnp.dot(h, w2_ref[0].astype(jnp.bfloat16),
                      preferred_element_type=jnp.float32)
        out_ref[...] = out * ws_ref[:, 0:1]

    @pl.when(e >= E)
    def _():
        out_ref[...] = jnp.zeros((C, D), jnp.float32)


def _run_ffn(counts, ein, w1, w2, wslot, interpret=False):
    nblk = NROWS // C
    grid = (nblk,)
    return pl.pallas_call(
        _ffn_body,
        grid=grid,
        in_specs=[
            pl.BlockSpec(memory_space=pltpu.SMEM),
            pl.BlockSpec((C, D // 2), lambda i: (i, 0)),
            pl.BlockSpec((1, D, F), lambda i: (jnp.minimum(i, E - 1), 0, 0)),
            pl.BlockSpec((1, F, D), lambda i: (jnp.minimum(i, E - 1), 0, 0)),
            pl.BlockSpec((C, 128), lambda i: (i, 0)),
        ],
        out_specs=pl.BlockSpec((C, D), lambda i: (i, 0)),
        out_shape=jax.ShapeDtypeStruct((NROWS, D), jnp.float32),
        interpret=interpret,
    )(counts, ein, w1, w2, wslot)


def _combine_body(outw_hbm, s_hbm, y_hbm, idx_v,
                  a0, b0_, a1, b1_,
                  sga0, sgb0, sga1, sgb1, swo0, swo1):
    cid = lax.axis_index("c")
    sid = lax.axis_index("s")
    w = sid * 2 + cid
    t0 = w * (T // NW)
    nsub = (T // NW) // DCH
    pltpu.sync_copy(s_hbm.at[w], idx_v)
    pa = (a0, a1)
    pb = (b0_, b1_)
    sga = (sga0, sga1)
    sgb = (sgb0, sgb1)
    swo = (swo0, swo1)

    def gathers(u):
        p = u % 2
        ga = pltpu.async_copy(
            outw_hbm.at[idx_v.at[0, pl.ds(u * DCH, DCH)]], pa[p], sga[p])
        gb = pltpu.async_copy(
            outw_hbm.at[idx_v.at[1, pl.ds(u * DCH, DCH)]], pb[p], sgb[p])
        return ga, gb

    g = gathers(0)
    wo = [None, None]
    for u in range(nsub):
        p = u % 2
        if u + 1 < nsub:
            if wo[(u + 1) % 2] is not None:
                wo[(u + 1) % 2].wait()
            gnext = gathers(u + 1)
        g[0].wait()
        g[1].wait()

        def rbody(r):
            for cc in range(D // 16):
                sl = pl.ds(cc * 16, 16)
                pa[p][r, sl] = pa[p][r, sl] + pb[p][r, sl]
        pl.loop(0, DCH)(rbody)
        wo[p] = pltpu.async_copy(pa[p], y_hbm.at[pl.ds(t0 + u * DCH, DCH)],
                                 swo[p])
        if u + 1 < nsub:
            g = gnext
    wo[0].wait()
    wo[1].wait()


def _run_combine(outw, s_d):
    mesh = plsc.VectorSubcoreMesh(core_axis_name="c", subcore_axis_name="s")
    kern = functools.partial(
        pl.kernel,
        out_type=jax.ShapeDtypeStruct((T, D), jnp.float32),
        mesh=mesh,
        scratch_types=[
            pltpu.VMEM((2, T // NW), jnp.int32),
            pltpu.VMEM((DCH, D), jnp.float32),
            pltpu.VMEM((DCH, D), jnp.float32),
            pltpu.VMEM((DCH, D), jnp.float32),
            pltpu.VMEM((DCH, D), jnp.float32),
            pltpu.SemaphoreType.DMA,
            pltpu.SemaphoreType.DMA,
            pltpu.SemaphoreType.DMA,
            pltpu.SemaphoreType.DMA,
            pltpu.SemaphoreType.DMA,
            pltpu.SemaphoreType.DMA,
        ],
    )
    return kern(_combine_body)(outw, s_d)


def kernel(x, w_gate, w1, w2):
    d0, d1, wrow, counts, aux, x16 = _run_router(x, w_gate)

    dk = jnp.stack([d0[:, 0], d1[:, 0]])
    d_b = dk.reshape(2, 16, 4, BCH).reshape(NW, 4, BCH)
    w_bb = wrow.reshape(2, 16, 4, BCH, 128).reshape(NW, 4, BCH, 128)
    ein, wslot = _run_dispatch(x16, d_b, w_bb)

    outw = _run_ffn(counts, ein, w1, w2, wslot)

    s_d = dk.reshape(2, NW, T // NW).transpose(1, 0, 2)
    y = _run_combine(outw, s_d)
    return y, aux[0, 0]


# speedup vs baseline: 1.6130x; 1.0371x over previous
import functools

import jax
import jax.numpy as jnp
from jax import lax
from jax.experimental import pallas as pl
from jax.experimental.pallas import tpu as pltpu
from jax.experimental.pallas import tpu_sc as plsc

E = 8
K = 2
D = 1024
F = 2048
T = 2048
C = 640
NROWS = (E + 1) * C
DUMP = E * C
LANES = 128
NW = 32
BCH = 32
DCH = 16


def _router_body(x_ref, wg_ref, dk_ref, wrow_ref,
                 counts_ref, aux_ref, x16_ref):
    x = x_ref[...]
    rt = x.astype(jnp.bfloat16).astype(jnp.float32)
    bits = lax.bitcast_convert_type(rt, jnp.uint32)
    lo = bits[:, :D // 2] >> 16
    hi = bits[:, D // 2:] & jnp.uint32(0xFFFF0000)
    x16_ref[...] = lax.bitcast_convert_type(lo | hi, jnp.int32)
    wg = wg_ref[...]
    logits = jnp.dot(x, wg, preferred_element_type=jnp.float32)
    lane = lax.broadcasted_iota(jnp.int32, (T, E), 1).astype(jnp.float32)
    m = jnp.max(logits, axis=1, keepdims=True)
    ex = jnp.exp(logits - m)
    z = jnp.sum(ex, axis=1, keepdims=True)
    probs = ex / z

    m1 = jnp.max(probs, axis=1, keepdims=True)
    is1 = probs == m1
    i1 = jnp.min(jnp.where(is1, lane, float(E)), axis=1, keepdims=True)
    mask0 = (lane == i1).astype(jnp.float32)
    p2 = jnp.where(mask0 > 0, -1.0, probs)
    m2 = jnp.max(p2, axis=1, keepdims=True)
    is2 = p2 == m2
    i2 = jnp.min(jnp.where(is2, lane, float(E)), axis=1, keepdims=True)
    mask1 = (lane == i2).astype(jnp.float32)

    denom = m1 + m2 + 1e-9
    g0 = m1 / denom
    g1 = m2 / denom

    B = 256
    r = lax.broadcasted_iota(jnp.int32, (B, B), 0)
    c = lax.broadcasted_iota(jnp.int32, (B, B), 1)
    ltri = (r > c).astype(jnp.float32)
    carry = jnp.zeros((1, E), dtype=jnp.float32)
    pos_parts = []
    for mask in (mask0, mask1):
        parts = []
        for b in range(T // B):
            mb = mask[b * B:(b + 1) * B, :]
            parts.append(jnp.dot(ltri, mb, preferred_element_type=jnp.float32)
                         + carry)
            carry = carry + jnp.sum(mb, axis=0, keepdims=True)
        pos_parts.append(jnp.concatenate(parts, axis=0))
    pos0, pos1 = pos_parts
    total = carry

    p0 = jnp.sum(pos0 * mask0, axis=1, keepdims=True)
    p1 = jnp.sum(pos1 * mask1, axis=1, keepdims=True)
    keep0 = p0 < float(C)
    keep1 = p1 < float(C)
    d0 = jnp.where(keep0, i1 * float(C) + p0, float(DUMP))
    d1 = jnp.where(keep1, i2 * float(C) + p1, float(DUMP))
    dk = jnp.transpose(jnp.concatenate([d0, d1], axis=1))
    dk_ref[...] = dk.astype(jnp.int32)
    w0e = jnp.where(keep0, g0, 0.0)
    w1e = jnp.where(keep1, g1, 0.0)
    w_all = jnp.concatenate([w0e, w1e], axis=0)
    wrow_ref[...] = jnp.broadcast_to(w_all, (K * T, 128))
    counts_ref[...] = jnp.concatenate(
        [jnp.minimum(total, float(C)),
         jnp.zeros((1, LANES - E), jnp.float32)], axis=1)

    em = jnp.maximum(mask0, mask1)
    tpe = jnp.sum(em, axis=0, keepdims=True) * (1.0 / T)
    ppe = jnp.sum(probs, axis=0, keepdims=True) * (1.0 / T)
    aux_ref[...] = jnp.sum(tpe * ppe, axis=1, keepdims=True) * float(E)


def _run_router(x, wg, interpret=False):
    out_shapes = (
        jax.ShapeDtypeStruct((K, T), jnp.int32),
        jax.ShapeDtypeStruct((K * T, 128), jnp.float32),
        jax.ShapeDtypeStruct((1, LANES), jnp.float32),
        jax.ShapeDtypeStruct((1, 1), jnp.float32),
        jax.ShapeDtypeStruct((T, D // 2), jnp.int32),
    )
    return pl.pallas_call(
        _router_body,
        out_shape=out_shapes,
        interpret=interpret,
    )(x, wg)


def _dispatch_body(x_hbm, d_hbm, wbb_hbm, ein_hbm, wslot_hbm,
                   idx_v, b0, b1, b2, wrows_v,
                   sa0, sa1, sa2, sb0, sb1, sb2, semw):
    cid = lax.axis_index("c")
    sid = lax.axis_index("s")
    w = sid * 2 + cid
    t0 = (w % 16) * (T // 16)
    k = w // 16
    bufs = (b0, b1, b2)
    sa = (sa0, sa1, sa2)
    sb = (sb0, sb1, sb2)
    stages = [pltpu.async_copy(x_hbm.at[pl.ds(t0 + j * BCH, BCH)],
                               bufs[j], sa[j]) for j in range(3)]
    for j in range(4):
        pltpu.sync_copy(d_hbm.at[k, pl.ds(t0 + j * BCH, BCH)], idx_v.at[j])
    pltpu.sync_copy(wbb_hbm.at[w], wrows_v)
    scat = [None] * 4
    wscat = []
    for j in range(4):
        if j == 3:
            scat[0].wait()
            stages.append(pltpu.async_copy(
                x_hbm.at[pl.ds(t0 + 3 * BCH, BCH)], b0, sa0))
        stages[j].wait()
        scat[j] = pltpu.async_copy(bufs[j % 3], ein_hbm.at[idx_v.at[j]],
                                   sb[j % 3])
        wscat.append(pltpu.async_copy(wrows_v.at[j],
                                      wslot_hbm.at[idx_v.at[j]], semw))
    for j in (1, 2, 3):
        scat[j].wait()
    for cp in wscat:
        cp.wait()


def _run_dispatch(x, d_b, w_bb):
    mesh = plsc.VectorSubcoreMesh(core_axis_name="c", subcore_axis_name="s")
    kern = functools.partial(
        pl.kernel,
        out_type=(
            jax.ShapeDtypeStruct((NROWS, D // 2), jnp.int32),
            jax.ShapeDtypeStruct((NROWS, 128), jnp.float32),
        ),
        mesh=mesh,
        scratch_types=[
            pltpu.VMEM((4, BCH), jnp.int32),
            pltpu.VMEM((BCH, D // 2), jnp.int32),
            pltpu.VMEM((BCH, D // 2), jnp.int32),
            pltpu.VMEM((BCH, D // 2), jnp.int32),
            pltpu.VMEM((4, BCH, 128), jnp.float32),
            pltpu.SemaphoreType.DMA,
            pltpu.SemaphoreType.DMA,
            pltpu.SemaphoreType.DMA,
            pltpu.SemaphoreType.DMA,
            pltpu.SemaphoreType.DMA,
            pltpu.SemaphoreType.DMA,
            pltpu.SemaphoreType.DMA,
        ],
    )
    return kern(_dispatch_body)(x, d_b, w_bb)


def _ffn_body(counts_ref, xin_ref, w1_ref, w2_ref, ws_ref, out_ref):
    e = pl.program_id(0)

    @pl.when(e < E)
    def _():
        cnt = counts_ref[0, e]
        row = lax.broadcasted_iota(jnp.int32, (C, 1), 0).astype(jnp.float32)
        u = lax.bitcast_convert_type(xin_ref[...], jnp.uint32)
        xlo = lax.bitcast_convert_type(u << 16, jnp.float32)
        xhi = lax.bitcast_convert_type(u & jnp.uint32(0xFFFF0000),
                                       jnp.float32)
        xp = jnp.concatenate([xlo, xhi], axis=1).astype(jnp.bfloat16)
        x = jnp.where(row < cnt, xp, jnp.zeros((C, D), jnp.bfloat16))
        h = jnp.dot(x, w1_ref[0].astype(jnp.bfloat16),
                    preferred_element_type=jnp.float32)
        h = jax.nn.gelu(h, approximate=True).astype(jnp.bfloat16)
        out = jnp.dot(h, w2_ref[0].astype(jnp.bfloat16),
                      preferred_element_type=jnp.float32)
        out_ref[...] = out * ws_ref[:, 0:1]

    @pl.when(e >= E)
    def _():
        out_ref[...] = jnp.zeros((C, D), jnp.float32)


def _run_ffn(counts, ein, w1, w2, wslot, interpret=False):
    nblk = NROWS // C
    grid = (nblk,)
    return pl.pallas_call(
        _ffn_body,
        grid=grid,
        in_specs=[
            pl.BlockSpec(memory_space=pltpu.SMEM),
            pl.BlockSpec((C, D // 2), lambda i: (i, 0)),
            pl.BlockSpec((1, D, F), lambda i: (jnp.minimum(i, E - 1), 0, 0)),
            pl.BlockSpec((1, F, D), lambda i: (jnp.minimum(i, E - 1), 0, 0)),
            pl.BlockSpec((C, 128), lambda i: (i, 0)),
        ],
        out_specs=pl.BlockSpec((C, D), lambda i: (i, 0)),
        out_shape=jax.ShapeDtypeStruct((NROWS, D), jnp.float32),
        interpret=interpret,
    )(counts, ein, w1, w2, wslot)


def _combine_body(outw_hbm, s_hbm, y_hbm, idx_v,
                  a0, b0_, a1, b1_,
                  sga0, sgb0, sga1, sgb1, swo0, swo1):
    cid = lax.axis_index("c")
    sid = lax.axis_index("s")
    w = sid * 2 + cid
    t0 = w * (T // NW)
    nsub = (T // NW) // DCH
    pltpu.sync_copy(s_hbm.at[0, pl.ds(t0, T // NW)], idx_v.at[0])
    pltpu.sync_copy(s_hbm.at[1, pl.ds(t0, T // NW)], idx_v.at[1])
    pa = (a0, a1)
    pb = (b0_, b1_)
    sga = (sga0, sga1)
    sgb = (sgb0, sgb1)
    swo = (swo0, swo1)

    def gathers(u):
        p = u % 2
        ga = pltpu.async_copy(
            outw_hbm.at[idx_v.at[0, pl.ds(u * DCH, DCH)]], pa[p], sga[p])
        gb = pltpu.async_copy(
            outw_hbm.at[idx_v.at[1, pl.ds(u * DCH, DCH)]], pb[p], sgb[p])
        return ga, gb

    g = gathers(0)
    wo = [None, None]
    for u in range(nsub):
        p = u % 2
        if u + 1 < nsub:
            if wo[(u + 1) % 2] is not None:
                wo[(u + 1) % 2].wait()
            gnext = gathers(u + 1)
        g[0].wait()
        g[1].wait()

        def rbody(r):
            for cc in range(D // 16):
                sl = pl.ds(cc * 16, 16)
                pa[p][r, sl] = pa[p][r, sl] + pb[p][r, sl]
        pl.loop(0, DCH)(rbody)
        wo[p] = pltpu.async_copy(pa[p], y_hbm.at[pl.ds(t0 + u * DCH, DCH)],
                                 swo[p])
        if u + 1 < nsub:
            g = gnext
    wo[0].wait()
    wo[1].wait()


def _run_combine(outw, s_d):
    mesh = plsc.VectorSubcoreMesh(core_axis_name="c", subcore_axis_name="s")
    kern = functools.partial(
        pl.kernel,
        out_type=jax.ShapeDtypeStruct((T, D), jnp.float32),
        mesh=mesh,
        scratch_types=[
            pltpu.VMEM((2, T // NW), jnp.int32),
            pltpu.VMEM((DCH, D), jnp.float32),
            pltpu.VMEM((DCH, D), jnp.float32),
            pltpu.VMEM((DCH, D), jnp.float32),
            pltpu.VMEM((DCH, D), jnp.float32),
            pltpu.SemaphoreType.DMA,
            pltpu.SemaphoreType.DMA,
            pltpu.SemaphoreType.DMA,
            pltpu.SemaphoreType.DMA,
            pltpu.SemaphoreType.DMA,
            pltpu.SemaphoreType.DMA,
        ],
    )
    return kern(_combine_body)(outw, s_d)


def kernel(x, w_gate, w1, w2):
    dk, wrow, counts, aux, x16 = _run_router(x, w_gate)

    w_bb = wrow.reshape(2, 16, 4, BCH, 128).reshape(NW, 4, BCH, 128)
    ein, wslot = _run_dispatch(x16, dk, w_bb)

    outw = _run_ffn(counts, ein, w1, w2, wslot)

    y = _run_combine(outw, dk)
    return y, aux[0, 0]
